# baseline (speedup 1.0000x reference)
import functools

import jax
import jax.numpy as jnp
from jax import lax
from jax.experimental import pallas as pl
from jax.experimental.pallas import tpu as pltpu
from jax.experimental.pallas import tpu_sc as plsc

S_LEN = 200
BATCH = 1024
N_TOK = S_LEN * BATCH
D_WORD = 64
D_TAG = 16
D_CROSS = 2 * D_TAG
D_OUT = D_WORD + D_CROSS
POS_DICT = 50
NER_DICT = 20

NUM_CORES = 2
NUM_SUBCORES = 16
NW = NUM_CORES * NUM_SUBCORES
TOK_PER_W = N_TOK // NW
SUB = 128
NCH = TOK_PER_W // SUB
NROWS = N_TOK // SUB
RING = 4
DEPTH = 4
CB = 2
LANES = 16
FB = D_OUT // 8
BB = BATCH // SUB


def _sc_embed(word_table, cross_table, idxw, idxp, idxn):
    mesh = plsc.VectorSubcoreMesh(core_axis_name="c", subcore_axis_name="s")

    @functools.partial(
        pl.kernel,
        out_type=jax.ShapeDtypeStruct((S_LEN, FB, BB, 8, SUB), jnp.float32),
        mesh=mesh,
        scratch_types=[
            pltpu.VMEM((NCH, SUB), jnp.int32),
            pltpu.VMEM((NCH, SUB), jnp.int32),
            pltpu.VMEM((NCH, SUB), jnp.int32),
            pltpu.VMEM((NCH, SUB), jnp.int32),
            pltpu.VMEM((RING, SUB, D_WORD), jnp.float32),
            pltpu.VMEM((RING, SUB, D_CROSS), jnp.float32),
            pltpu.VMEM((CB, FB, 1, 8, SUB), jnp.float32),
            pltpu.SemaphoreType.DMA,
            pltpu.SemaphoreType.DMA,
        ],
        compiler_params=pltpu.CompilerParams(
            use_tc_tiling_on_sc=False, needs_layout_passes=False
        ),
    )
    def k(wt, ct, iw, ip, inr, out, iw_v, ip_v, in_v, it_v, wbuf, tbuf, cbuf,
          gsem, wsem):
        wid = lax.axis_index("s") * NUM_CORES + lax.axis_index("c")
        row0 = wid * NCH

        pltpu.sync_copy(iw.at[pl.ds(row0, NCH)], iw_v)
        pltpu.sync_copy(ip.at[pl.ds(row0, NCH)], ip_v)
        pltpu.sync_copy(inr.at[pl.ds(row0, NCH)], in_v)

        def tag_body(r, c):
            for g in range(SUB // LANES):
                p = ip_v[r, pl.ds(g * LANES, LANES)]
                n = in_v[r, pl.ds(g * LANES, LANES)]
                it_v[r, pl.ds(g * LANES, LANES)] = p * NER_DICT + n
            return c

        lax.fori_loop(0, NCH, tag_body, 0)

        def fire(cg, slot):
            pltpu.make_async_copy(wt.at[iw_v.at[cg]], wbuf.at[slot], gsem).start()
            pltpu.make_async_copy(ct.at[it_v.at[cg]], tbuf.at[slot], gsem).start()

        def write_desc(cslot, sg, bblk):
            return pltpu.make_async_copy(
                cbuf.at[cslot],
                out.at[sg, pl.ds(0, FB), pl.ds(bblk, 1)],
                wsem,
            )

        for cg in range(DEPTH):
            fire(cg, cg)

        iota = lax.iota(jnp.int32, LANES)

        def body(ci, c):
            gslot = lax.rem(ci, RING)
            cslot = lax.rem(ci, CB)
            g = row0 + ci
            sg = lax.div(g, BB)
            bblk = lax.rem(g, BB)

            pltpu.make_async_copy(wt.at[iw_v.at[ci]], wbuf.at[gslot], gsem).wait()
            pltpu.make_async_copy(ct.at[it_v.at[ci]], tbuf.at[gslot], gsem).wait()

            @pl.when(ci >= CB)
            def _():
                write_desc(cslot, sg, bblk).wait()

            def tb_body(tb, c2):
                rows = tb * LANES + iota
                for f in range(D_OUT):
                    if f < D_WORD:
                        src, col = wbuf, f
                    else:
                        src, col = tbuf, f - D_WORD
                    cols = jnp.full((LANES,), col, jnp.int32)
                    v = plsc.load_gather(src.at[gslot], [rows, cols])
                    cbuf[cslot, f // 8, 0, f % 8, pl.ds(tb * LANES, LANES)] = v
                return c2

            lax.fori_loop(0, SUB // LANES, tb_body, 0)

            write_desc(cslot, sg, bblk).start()

            @pl.when(ci + DEPTH < NCH)
            def _():
                fire(ci + DEPTH, lax.rem(ci + DEPTH, RING))
            return c

        lax.fori_loop(0, NCH, body, 0)

        for _i in range(CB):
            write_desc(0, 0, 0).wait()

    return k(word_table, cross_table, idxw, idxp, idxn)


def kernel(seq_word, seq_pos, seq_ner, word_table, pos_table, ner_table):
    cross = jnp.concatenate(
        [
            jnp.broadcast_to(pos_table[:, None, :], (POS_DICT, NER_DICT, D_TAG)),
            jnp.broadcast_to(ner_table[None, :, :], (POS_DICT, NER_DICT, D_TAG)),
        ],
        axis=2,
    ).reshape(POS_DICT * NER_DICT, D_CROSS)
    idxw = seq_word.reshape(NROWS, SUB).astype(jnp.int32)
    idxp = seq_pos.reshape(NROWS, SUB).astype(jnp.int32)
    idxn = seq_ner.reshape(NROWS, SUB).astype(jnp.int32)
    out5 = _sc_embed(word_table, cross, idxw, idxp, idxn)
    return (
        out5.transpose(0, 2, 4, 1, 3).reshape(S_LEN, BATCH, D_OUT)
    )
